# baseline (speedup 1.0000x reference)
import functools

import jax
import jax.numpy as jnp
from jax.experimental import pallas as pl

N = 11616
NHID = 128
ROW_BLK = 264
N_BLKS = N // ROW_BLK
GAMMA = 2.0
_INTERPRET = False


def _mm_bias_body(x_ref, w_ref, b_ref, o_ref):
    o_ref[...] = (
        jnp.dot(x_ref[...], w_ref[...], preferred_element_type=jnp.float32)
        + b_ref[...]
    )


def _mm_bias(x, w, b):
    m, _ = x.shape
    _, n = w.shape
    return pl.pallas_call(
        _mm_bias_body,
        out_shape=jax.ShapeDtypeStruct((m, n), jnp.float32),
        interpret=_INTERPRET,
    )(x, w, b.reshape(1, n))


def _gcn_body(relu, residual, a_ref, y_ref, b_ref, x_ref, o_ref):
    s = jnp.dot(a_ref[...], y_ref[...], preferred_element_type=jnp.float32)
    s = s + b_ref[...]
    if relu:
        s = jnp.maximum(s, 0.0)
    if residual:
        s = s + x_ref[...]
    o_ref[...] = s


def _gcn_layer(A, x, W, b, relu, residual):
    y = _mm_bias(x, W, jnp.zeros((NHID,), jnp.float32))
    body = functools.partial(_gcn_body, relu, residual)
    return pl.pallas_call(
        body,
        grid=(N_BLKS,),
        in_specs=[
            pl.BlockSpec((ROW_BLK, N), lambda i: (i, 0)),
            pl.BlockSpec((N, NHID), lambda i: (0, 0)),
            pl.BlockSpec((1, NHID), lambda i: (0, 0)),
            pl.BlockSpec((ROW_BLK, NHID), lambda i: (i, 0)),
        ],
        out_specs=pl.BlockSpec((ROW_BLK, NHID), lambda i: (i, 0)),
        out_shape=jax.ShapeDtypeStruct((N, NHID), jnp.float32),
        interpret=_INTERPRET,
    )(A, y, b.reshape(1, NHID), x)


def _bce_body(x_blk_ref, x_all_ref, adj_ref, o_ref):
    i = pl.program_id(0)
    z = jax.lax.dot_general(
        x_blk_ref[...], x_all_ref[...],
        (((1,), (1,)), ((), ())),
        preferred_element_type=jnp.float32,
    )
    p = jax.nn.sigmoid(z)
    p = jnp.clip(p, 1e-7, 1.0 - 1e-7)
    a = adj_ref[...]
    ll = a * jnp.log(p) + (1.0 - a) * jnp.log(1.0 - p)
    part = -jnp.sum(ll).reshape(1, 1)

    @pl.when(i == 0)
    def _():
        o_ref[...] = jnp.zeros((1, 1), jnp.float32)

    o_ref[...] += part


def _bce_loss(x, adj_full):
    s = pl.pallas_call(
        _bce_body,
        grid=(N_BLKS,),
        in_specs=[
            pl.BlockSpec((ROW_BLK, NHID), lambda i: (i, 0)),
            pl.BlockSpec((N, NHID), lambda i: (0, 0)),
            pl.BlockSpec((ROW_BLK, N), lambda i: (i, 0)),
        ],
        out_specs=pl.BlockSpec((1, 1), lambda i: (0, 0)),
        out_shape=jax.ShapeDtypeStruct((1, 1), jnp.float32),
        interpret=_INTERPRET,
    )(x, x, adj_full)
    return s[0, 0] / (float(N) * float(N))


def _recon_loss_body(r_ref, f_ref, o_ref):
    r = r_ref[...]
    f = f_ref[...]
    rn = jnp.maximum(jnp.sqrt(jnp.sum(r * r, axis=-1, keepdims=True)), 1e-12)
    fn = jnp.maximum(jnp.sqrt(jnp.sum(f * f, axis=-1, keepdims=True)), 1e-12)
    cs = jnp.sum((r / rn) * (f / fn), axis=-1)
    o_ref[...] = jnp.sum((1.0 - cs) ** GAMMA).reshape(1, 1)


def _recon_loss(recon0, X0):
    s = pl.pallas_call(
        _recon_loss_body,
        out_shape=jax.ShapeDtypeStruct((1, 1), jnp.float32),
        interpret=_INTERPRET,
    )(recon0, X0)
    return s[0, 0] / float(X0.shape[0])


def kernel(A, adj_full, X0, X1, X2, fcW0, fcb0, fcW1, fcb1, fcW2, fcb2,
           encW0, encb0, encW1, encb1, decW0, decb0, decW1, decb1,
           fc2W0, fc2b0, fc2W1, fc2b1, fc2W2, fc2b2):
    trans = [
        _mm_bias(X0, fcW0, fcb0),
        _mm_bias(X1, fcW1, fcb1),
        _mm_bias(X2, fcW2, fcb2),
    ]
    x = jnp.concatenate(trans, axis=0)

    x = _gcn_layer(A, x, encW0, encb0, relu=True, residual=False)
    x = _gcn_layer(A, x, encW1, encb1, relu=False, residual=True)
    x = _gcn_layer(A, x, decW0, decb0, relu=True, residual=False)
    x = _gcn_layer(A, x, decW1, decb1, relu=False, residual=True)

    n0, n1 = X0.shape[0], X1.shape[0]
    emb0 = x[:n0]
    emb1 = x[n0:n0 + n1]
    emb2 = x[n0 + n1:]
    recon0 = _mm_bias(emb0, fc2W0, fc2b0)
    recon1 = _mm_bias(emb1, fc2W1, fc2b1)
    recon2 = _mm_bias(emb2, fc2W2, fc2b2)

    emb_loss = _bce_loss(x, adj_full)
    recon_loss = _recon_loss(recon0, X0)
    return (recon0, recon1, recon2, emb_loss, recon_loss)
